# Initial kernel scaffold; baseline (speedup 1.0000x reference)
import functools

import jax
import jax.numpy as jnp
from jax.experimental import pallas as pl
from jax.experimental.pallas import tpu as pltpu
from jax.experimental.pallas import tpu_sc as plsc

_LT = 2048
_VT = 2048


def _im2col(x, K, stride, pad):
    B, L, C = x.shape
    xp = jnp.pad(x, ((0, 0), (pad, pad), (0, 0)))
    Lo = L // stride
    cols = [xp[:, k:k + stride * Lo:stride, :] for k in range(K)]
    return jnp.concatenate(cols, axis=2)


def _mm_bias_act(x, w, b, relu):
    m, _ = x.shape
    n = w.shape[1]

    def body(x_ref, w_ref, b_ref, o_ref):
        y = jax.lax.dot_general(
            x_ref[...], w_ref[...], (((1,), (0,)), ((), ())),
            preferred_element_type=jnp.float32)
        y = y + b_ref[...]
        if relu:
            y = jnp.maximum(y, 0.0)
        o_ref[...] = y

    return pl.pallas_call(
        body,
        out_shape=jax.ShapeDtypeStruct((m, n), jnp.float32),
    )(x, w, b.reshape(1, n))


def _conv_stack(audio, w1, b1, w2, b2, w3, b3):
    B = audio.shape[0]
    L = audio.shape[2]
    x = jnp.transpose(audio, (0, 2, 1))
    X1 = _im2col(x, 7, 2, 3)
    h1 = _mm_bias_act(X1.reshape(B * (L // 2), 7),
                      jnp.transpose(w1, (2, 1, 0)).reshape(7, -1), b1, True)
    c1 = h1.shape[1]
    X2 = _im2col(h1.reshape(B, L // 2, c1), 7, 2, 3)
    h2 = _mm_bias_act(X2.reshape(B * (L // 4), 7 * c1),
                      jnp.transpose(w2, (2, 1, 0)).reshape(7 * c1, -1), b2, True)
    c2 = h2.shape[1]
    X3 = _im2col(h2.reshape(B, L // 4, c2), 7, 2, 3)
    f = _mm_bias_act(X3.reshape(B * (L // 8), 7 * c2),
                     jnp.transpose(w3, (2, 1, 0)).reshape(7 * c2, -1), b3, False)
    return f


def _vq_tokens(f, cbT):
    BL, D = f.shape
    ncb, _, V = cbT.shape
    nl = BL // _LT
    nv = V // _VT

    def body(f_ref, cb_ref, tok_ref, minv, argm):
        j = pl.program_id(2)

        @pl.when(j == 0)
        def _init():
            minv[...] = jnp.full(minv.shape, jnp.inf, jnp.float32)
            argm[...] = jnp.zeros(argm.shape, jnp.int32)

        fv = f_ref[...]
        cb = cb_ref[0]
        f2 = jnp.sum(fv * fv, axis=1, keepdims=True)
        c2 = jnp.sum(cb * cb, axis=0, keepdims=True)
        e = jax.lax.dot_general(
            fv, cb, (((1,), (0,)), ((), ())),
            preferred_element_type=jnp.float32)
        scores = (f2 + c2) - 2.0 * e
        tmin = jnp.min(scores, axis=1, keepdims=True)
        idx = jax.lax.broadcasted_iota(jnp.int32, scores.shape, 1) + j * _VT
        targ = jnp.min(jnp.where(scores == tmin, idx, jnp.int32(2 ** 30)),
                       axis=1, keepdims=True)
        better = tmin < minv[...]
        argm[...] = jnp.where(better, targ, argm[...])
        minv[...] = jnp.where(better, tmin, minv[...])

        @pl.when(j == nv - 1)
        def _fin():
            tok_ref[0] = argm[...]

    return pl.pallas_call(
        body,
        grid=(ncb, nl, nv),
        in_specs=[
            pl.BlockSpec((_LT, D), lambda i, l, j: (l, 0)),
            pl.BlockSpec((1, D, _VT), lambda i, l, j: (i, 0, j)),
        ],
        out_specs=pl.BlockSpec((1, _LT, 1), lambda i, l, j: (i, l, 0)),
        out_shape=jax.ShapeDtypeStruct((ncb, BL, 1), jnp.int32),
        scratch_shapes=[
            pltpu.VMEM((_LT, 1), jnp.float32),
            pltpu.VMEM((_LT, 1), jnp.int32),
        ],
    )(f, cbT)


def _sc_gather_mean(tokens, emb_table):
    ncb, BL = tokens.shape
    D = emb_table.shape[1]
    NC, NS = 2, 16
    NW = NC * NS
    CHUNK = BL // NW
    mesh = plsc.VectorSubcoreMesh(core_axis_name="c", subcore_axis_name="s")

    @functools.partial(
        pl.kernel, mesh=mesh,
        out_type=jax.ShapeDtypeStruct((BL, D), jnp.float32),
        scratch_types=(
            [pltpu.VMEM((CHUNK,), jnp.int32) for _ in range(ncb)]
            + [pltpu.VMEM((CHUNK, D), jnp.float32) for _ in range(ncb)]
            + [pltpu.VMEM((CHUNK, D), jnp.float32), pltpu.SemaphoreType.DMA]
        ),
    )
    def gather_kernel(tok_hbm, emb_hbm, out_hbm, *scratch):
        idxs = scratch[:ncb]
        rows = scratch[ncb:2 * ncb]
        acc = scratch[2 * ncb]
        sem = scratch[2 * ncb + 1]
        wid = jax.lax.axis_index("s") * NC + jax.lax.axis_index("c")
        base = wid * CHUNK
        for cb in range(ncb):
            pltpu.sync_copy(tok_hbm.at[cb, pl.ds(base, CHUNK)], idxs[cb])
        copies = [pltpu.async_copy(emb_hbm.at[idxs[cb]], rows[cb], sem)
                  for cb in range(ncb)]
        for cp in copies:
            cp.wait()
        scale = 1.0 / ncb

        def body(i, carry):
            for c0 in range(0, D, 16):
                s = rows[0][i, pl.ds(c0, 16)]
                for cb in range(1, ncb):
                    s = s + rows[cb][i, pl.ds(c0, 16)]
                acc[i, pl.ds(c0, 16)] = s * scale
            return carry

        jax.lax.fori_loop(0, CHUNK, body, 0)
        pltpu.sync_copy(acc, out_hbm.at[pl.ds(base, CHUNK)])

    return gather_kernel(tokens, emb_table)


def kernel(audio, w1, b1, w2, b2, w3, b3, codebook, emb_table):
    B = audio.shape[0]
    ncb = codebook.shape[0]
    f = _conv_stack(audio, w1, b1, w2, b2, w3, b3)
    cbT = jnp.transpose(codebook, (0, 2, 1))
    toks = _vq_tokens(f, cbT)[..., 0]
    emb = _sc_gather_mean(toks, emb_table)
    L3 = toks.shape[1] // B
    tokens = jnp.transpose(toks.reshape(ncb, B, L3), (1, 0, 2))
    embeddings = emb.reshape(B, L3, emb.shape[1])
    return tokens, embeddings


# speedup vs baseline: 1.1467x; 1.1467x over previous
import functools

import jax
import jax.numpy as jnp
from jax.experimental import pallas as pl
from jax.experimental.pallas import tpu as pltpu
from jax.experimental.pallas import tpu_sc as plsc

_LT = 2048
_VT = 2048


def _im2col(x, K, stride, pad):
    B, L, C = x.shape
    xp = jnp.pad(x, ((0, 0), (pad, pad), (0, 0)))
    Lo = L // stride
    cols = [xp[:, k:k + stride * Lo:stride, :] for k in range(K)]
    return jnp.concatenate(cols, axis=2)


def _mm_bias_act(x, w, b, relu):
    m, _ = x.shape
    n = w.shape[1]

    def body(x_ref, w_ref, b_ref, o_ref):
        y = jax.lax.dot_general(
            x_ref[...], w_ref[...], (((1,), (0,)), ((), ())),
            preferred_element_type=jnp.float32)
        y = y + b_ref[...]
        if relu:
            y = jnp.maximum(y, 0.0)
        o_ref[...] = y

    return pl.pallas_call(
        body,
        out_shape=jax.ShapeDtypeStruct((m, n), jnp.float32),
    )(x, w, b.reshape(1, n))


def _conv_stack(audio, w1, b1, w2, b2, w3, b3):
    B = audio.shape[0]
    L = audio.shape[2]
    x = jnp.transpose(audio, (0, 2, 1))
    X1 = _im2col(x, 7, 2, 3)
    h1 = _mm_bias_act(X1.reshape(B * (L // 2), 7),
                      jnp.transpose(w1, (2, 1, 0)).reshape(7, -1), b1, True)
    c1 = h1.shape[1]
    X2 = _im2col(h1.reshape(B, L // 2, c1), 7, 2, 3)
    h2 = _mm_bias_act(X2.reshape(B * (L // 4), 7 * c1),
                      jnp.transpose(w2, (2, 1, 0)).reshape(7 * c1, -1), b2, True)
    c2 = h2.shape[1]
    X3 = _im2col(h2.reshape(B, L // 4, c2), 7, 2, 3)
    f = _mm_bias_act(X3.reshape(B * (L // 8), 7 * c2),
                     jnp.transpose(w3, (2, 1, 0)).reshape(7 * c2, -1), b3, False)
    return f


def _vq_tokens(f, cbT):
    BL, D = f.shape
    ncb, _, V = cbT.shape
    nl = BL // _LT
    nv = V // _VT

    def body(f_ref, cb_ref, tok_ref, minv, argm):
        j = pl.program_id(2)

        @pl.when(j == 0)
        def _init():
            minv[...] = jnp.full(minv.shape, jnp.inf, jnp.float32)
            argm[...] = jnp.zeros(argm.shape, jnp.int32)

        fv = f_ref[...]
        cb = cb_ref[0]
        f2 = jnp.sum(fv * fv, axis=1, keepdims=True)
        c2 = jnp.sum(cb * cb, axis=0, keepdims=True)
        e = jax.lax.dot_general(
            fv, cb, (((1,), (0,)), ((), ())),
            preferred_element_type=jnp.float32)
        scores = (f2 + c2) - 2.0 * e
        tmin = jnp.min(scores, axis=1, keepdims=True)
        idx = jax.lax.broadcasted_iota(jnp.int32, scores.shape, 1) + j * _VT
        targ = jnp.min(jnp.where(scores == tmin, idx, jnp.int32(2 ** 30)),
                       axis=1, keepdims=True)
        better = tmin < minv[...]
        argm[...] = jnp.where(better, targ, argm[...])
        minv[...] = jnp.where(better, tmin, minv[...])

        @pl.when(j == nv - 1)
        def _fin():
            tok_ref[0] = argm[...]

    return pl.pallas_call(
        body,
        grid=(ncb, nl, nv),
        in_specs=[
            pl.BlockSpec((_LT, D), lambda i, l, j: (l, 0)),
            pl.BlockSpec((1, D, _VT), lambda i, l, j: (i, 0, j)),
        ],
        out_specs=pl.BlockSpec((1, _LT, 1), lambda i, l, j: (i, l, 0)),
        out_shape=jax.ShapeDtypeStruct((ncb, BL, 1), jnp.int32),
        scratch_shapes=[
            pltpu.VMEM((_LT, 1), jnp.float32),
            pltpu.VMEM((_LT, 1), jnp.int32),
        ],
    )(f, cbT)


def _sc_gather_mean(tokens, emb_table):
    ncb, BL = tokens.shape
    D = emb_table.shape[1]
    DP = 128
    emb_table = jnp.pad(emb_table, ((0, 0), (0, DP - D)))
    NC, NS = 2, 16
    NW = NC * NS
    CHUNK = BL // NW
    mesh = plsc.VectorSubcoreMesh(core_axis_name="c", subcore_axis_name="s")

    @functools.partial(
        pl.kernel, mesh=mesh,
        out_type=jax.ShapeDtypeStruct((BL, D), jnp.float32),
        scratch_types=(
            [pltpu.VMEM((CHUNK,), jnp.int32) for _ in range(ncb)]
            + [pltpu.VMEM((CHUNK, DP), jnp.float32) for _ in range(ncb)]
            + [pltpu.VMEM((CHUNK, D), jnp.float32), pltpu.SemaphoreType.DMA]
        ),
    )
    def gather_kernel(tok_hbm, emb_hbm, out_hbm, *scratch):
        idxs = scratch[:ncb]
        rows = scratch[ncb:2 * ncb]
        acc = scratch[2 * ncb]
        sem = scratch[2 * ncb + 1]
        wid = jax.lax.axis_index("s") * NC + jax.lax.axis_index("c")
        base = wid * CHUNK
        for cb in range(ncb):
            pltpu.sync_copy(tok_hbm.at[cb, pl.ds(base, CHUNK)], idxs[cb])
        copies = [pltpu.async_copy(emb_hbm.at[idxs[cb]], rows[cb], sem)
                  for cb in range(ncb)]
        for cp in copies:
            cp.wait()
        scale = 1.0 / ncb

        def body(i, carry):
            for c0 in range(0, D, 16):
                s = rows[0][i, pl.ds(c0, 16)]
                for cb in range(1, ncb):
                    s = s + rows[cb][i, pl.ds(c0, 16)]
                acc[i, pl.ds(c0, 16)] = s * scale
            return carry

        jax.lax.fori_loop(0, CHUNK, body, 0)
        pltpu.sync_copy(acc, out_hbm.at[pl.ds(base, CHUNK)])

    return gather_kernel(tokens, emb_table)


def kernel(audio, w1, b1, w2, b2, w3, b3, codebook, emb_table):
    B = audio.shape[0]
    ncb = codebook.shape[0]
    f = _conv_stack(audio, w1, b1, w2, b2, w3, b3)
    cbT = jnp.transpose(codebook, (0, 2, 1))
    toks = _vq_tokens(f, cbT)[..., 0]
    emb = _sc_gather_mean(toks, emb_table)
    L3 = toks.shape[1] // B
    tokens = jnp.transpose(toks.reshape(ncb, B, L3), (1, 0, 2))
    embeddings = emb.reshape(B, L3, emb.shape[1])
    return tokens, embeddings
